# baseline (speedup 1.0000x reference)
import functools

import jax
import jax.numpy as jnp
from jax import lax
from jax.experimental import pallas as pl
from jax.experimental.pallas import tpu as pltpu
from jax.experimental.pallas import tpu_sc as plsc

B, C, H, W = 16, 96, 224, 224
NC, NS, L = 2, 16, 16
NW = NC * NS
CQ = C // 4
UNITS = 14
NBLK = UNITS * 4
W0 = 128
W1 = W - W0

_mesh = plsc.VectorSubcoreMesh(
    core_axis_name="c", subcore_axis_name="s", num_cores=NC, num_subcores=NS
)


@functools.partial(
    pl.kernel,
    out_type=jax.ShapeDtypeStruct((NW, 2 * L), jnp.float32),
    mesh=_mesh,
    compiler_params=pltpu.CompilerParams(
        use_tc_tiling_on_sc=True, needs_layout_passes=False
    ),
    scratch_types=[
        pltpu.VMEM((CQ, 8, W0), jnp.float32),
        pltpu.VMEM((CQ, 8, W0), jnp.float32),
        pltpu.VMEM((CQ, 8, W1), jnp.float32),
        pltpu.VMEM((CQ, 8, W1), jnp.float32),
        pltpu.VMEM((8, W), jnp.int32),
        pltpu.VMEM((8, W), jnp.float32),
        pltpu.VMEM((2 * L,), jnp.float32),
        pltpu.SemaphoreType.DMA,
        pltpu.SemaphoreType.DMA,
    ],
)
def _traloss_sc(yhat_hbm, label_hbm, mask_hbm, parts_hbm,
                a0_v, a1_v, b0_v, b1_v, lab_v, m_v, part_v, sem0, sem1):
    c = lax.axis_index("c")
    s = lax.axis_index("s")
    wid = c * NS + s
    b = lax.shift_right_logical(wid, 1)
    hb0 = UNITS * (wid & 1)
    lane = lax.iota(jnp.int32, L)
    zero = jnp.zeros((L,), jnp.float32)

    def make_pass(w_start, w_width):
        k_lo, k_hi = w_start // L, (w_start + w_width) // L

        def src(g):
            h8 = (hb0 + lax.shift_right_logical(g, 2)) * 8
            c0 = CQ * (g & 3)
            return yhat_hbm.at[
                b, pl.ds(c0, CQ), pl.ds(h8, 8), pl.ds(w_start, w_width)
            ]

        def fire(g, buf, sem):
            pltpu.async_copy(src(g), buf, sem)

        def drain(g, buf, sem):
            pltpu.make_async_copy(src(g), buf, sem).wait()

        def load_labels(g):
            h8 = (hb0 + lax.shift_right_logical(g, 2)) * 8
            pltpu.sync_copy(label_hbm.at[b, 0, pl.ds(h8, 8)], lab_v)
            pltpu.sync_copy(mask_hbm.at[b, 0, pl.ds(h8, 8)], m_v)

        def compute(g, buf, carry):
            an, ad = carry
            c0 = CQ * (g & 3)
            first_q = (g & 3) == 0
            for r in range(8):
                for k in range(k_lo, k_hi):
                    cs = k * L
                    lab = lab_v[r, pl.ds(cs, L)]
                    cc = lab - c0
                    inr = (cc >= 0) & (cc < CQ)
                    cidx = jnp.clip(cc, 0, CQ - 1)
                    hidx = jnp.full((L,), r, jnp.int32)
                    widx = (cs - w_start) + lane
                    g_val = plsc.load_gather(buf, [cidx, hidx, widx])
                    m = m_v[r, pl.ds(cs, L)]
                    an = an + jnp.where(inr, g_val, 0.0) * m
                    ad = ad + jnp.where(first_q, m, 0.0)
            return an, ad

        return fire, drain, load_labels, compute

    def run_pass(carry, w_start, w_width, buf0, buf1):
        fire, drain, load_labels, compute = make_pass(w_start, w_width)
        fire(0, buf0, sem0)

        def pair_body(j, carry):
            g0 = 2 * j
            fire(g0 + 1, buf1, sem1)
            load_labels(g0)
            drain(g0, buf0, sem0)
            carry = compute(g0, buf0, carry)

            @pl.when(g0 + 2 < NBLK)
            def _():
                fire(g0 + 2, buf0, sem0)

            drain(g0 + 1, buf1, sem1)
            carry = compute(g0 + 1, buf1, carry)
            return carry

        return lax.fori_loop(0, NBLK // 2, pair_body, carry)

    carry = run_pass((zero, zero), 0, W0, a0_v, a1_v)
    an, ad = run_pass(carry, W0, W1, b0_v, b1_v)

    part_v[pl.ds(0, L)] = an
    part_v[pl.ds(L, L)] = ad
    pltpu.sync_copy(part_v, parts_hbm.at[wid])


def _fold_body(parts_ref, o_ref):
    x = parts_ref[...]
    rn = jnp.sum(x[:, :L], axis=1)
    rd = jnp.sum(x[:, L:], axis=1)
    row = lax.broadcasted_iota(jnp.int32, (B, NW), 1)
    samp = lax.broadcasted_iota(jnp.int32, (B, NW), 0)
    sel = jnp.where(row // 2 == samp, 1.0, 0.0)
    num = jnp.sum(sel * rn[None, :], axis=1)
    den = jnp.sum(sel * rd[None, :], axis=1)
    o_ref[0, 0] = jnp.sum(num / den) * (1.0 / B)


_fold = pl.pallas_call(
    _fold_body,
    out_shape=jax.ShapeDtypeStruct((1, 1), jnp.float32),
    out_specs=pl.BlockSpec(memory_space=pltpu.SMEM),
)


def kernel(y_hat, label, mask):
    parts = _traloss_sc(y_hat, label.astype(jnp.int32), mask)
    return _fold(parts)[0, 0]


# speedup vs baseline: 1.0490x; 1.0490x over previous
import functools

import jax
import jax.numpy as jnp
from jax import lax
from jax.experimental import pallas as pl
from jax.experimental.pallas import tpu as pltpu
from jax.experimental.pallas import tpu_sc as plsc

B, C, H, W = 16, 96, 224, 224
NC, NS, L = 2, 16, 16
NW = NC * NS
CB = C // 8
UNITS = 14
HROWS = UNITS * 8
NBLK = UNITS * 8
VPR_W = W // L

_mesh = plsc.VectorSubcoreMesh(
    core_axis_name="c", subcore_axis_name="s", num_cores=NC, num_subcores=NS
)


@functools.partial(
    pl.kernel,
    out_type=jax.ShapeDtypeStruct((NW, 2 * L), jnp.float32),
    mesh=_mesh,
    compiler_params=pltpu.CompilerParams(
        use_tc_tiling_on_sc=True, needs_layout_passes=False
    ),
    scratch_types=[
        pltpu.VMEM((CB, 8, W), jnp.float32),
        pltpu.VMEM((CB, 8, W), jnp.float32),
        pltpu.VMEM((HROWS, W), jnp.int32),
        pltpu.VMEM((HROWS, W), jnp.float32),
        pltpu.VMEM((2 * L,), jnp.float32),
        pltpu.SemaphoreType.DMA,
        pltpu.SemaphoreType.DMA,
    ],
)
def _traloss_sc(yhat_hbm, label_hbm, mask_hbm, parts_hbm,
                y0_v, y1_v, lab_v, m_v, part_v, sem0, sem1):
    c = lax.axis_index("c")
    s = lax.axis_index("s")
    wid = c * NS + s
    b = lax.shift_right_logical(wid, 1)
    h0 = HROWS * (wid & 1)
    lane = lax.iota(jnp.int32, L)
    zero = jnp.zeros((L,), jnp.float32)

    pltpu.sync_copy(label_hbm.at[b, 0, pl.ds(h0, HROWS)], lab_v)
    pltpu.sync_copy(mask_hbm.at[b, 0, pl.ds(h0, HROWS)], m_v)

    def ad_body(row, ad):
        for k in range(VPR_W):
            ad = ad + m_v[row, pl.ds(k * L, L)]
        return ad

    ad = lax.fori_loop(0, HROWS, ad_body, zero)

    def src(g):
        h8 = h0 + lax.shift_right_logical(g, 3) * 8
        c0 = CB * (g & 7)
        return yhat_hbm.at[b, pl.ds(c0, CB), pl.ds(h8, 8)]

    def fire(g, buf, sem):
        pltpu.async_copy(src(g), buf, sem)

    def drain(g, buf, sem):
        pltpu.make_async_copy(src(g), buf, sem).wait()

    def compute(g, buf, an):
        row0 = lax.shift_right_logical(g, 3) * 8
        c0 = CB * (g & 7)
        for r in range(8):
            row = row0 + r
            for k in range(VPR_W):
                cs = k * L
                lab = lab_v[row, pl.ds(cs, L)]
                cc = lab - c0
                inr = (cc >= 0) & (cc < CB)
                cidx = jnp.clip(cc, 0, CB - 1)
                hidx = jnp.full((L,), r, jnp.int32)
                widx = cs + lane
                g_val = plsc.load_gather(buf, [cidx, hidx, widx])
                an = an + jnp.where(inr, g_val, 0.0) * m_v[row, pl.ds(cs, L)]
        return an

    fire(0, y0_v, sem0)

    def pair_body(j, an):
        g0 = 2 * j
        fire(g0 + 1, y1_v, sem1)
        drain(g0, y0_v, sem0)
        an = compute(g0, y0_v, an)

        @pl.when(g0 + 2 < NBLK)
        def _():
            fire(g0 + 2, y0_v, sem0)

        drain(g0 + 1, y1_v, sem1)
        an = compute(g0 + 1, y1_v, an)
        return an

    an = lax.fori_loop(0, NBLK // 2, pair_body, zero)

    part_v[pl.ds(0, L)] = an
    part_v[pl.ds(L, L)] = ad
    pltpu.sync_copy(part_v, parts_hbm.at[wid])


def _fold_body(parts_ref, o_ref):
    x = parts_ref[...]
    rn = jnp.sum(x[:, :L], axis=1)
    rd = jnp.sum(x[:, L:], axis=1)
    row = lax.broadcasted_iota(jnp.int32, (B, NW), 1)
    samp = lax.broadcasted_iota(jnp.int32, (B, NW), 0)
    sel = jnp.where(row // 2 == samp, 1.0, 0.0)
    num = jnp.sum(sel * rn[None, :], axis=1)
    den = jnp.sum(sel * rd[None, :], axis=1)
    o_ref[0, 0] = jnp.sum(num / den) * (1.0 / B)


_fold = pl.pallas_call(
    _fold_body,
    out_shape=jax.ShapeDtypeStruct((1, 1), jnp.float32),
    out_specs=pl.BlockSpec(memory_space=pltpu.SMEM),
)


def kernel(y_hat, label, mask):
    parts = _traloss_sc(y_hat, label.astype(jnp.int32), mask)
    return _fold(parts)[0, 0]


# speedup vs baseline: 1.1927x; 1.1370x over previous
import functools

import jax
import jax.numpy as jnp
from jax import lax
from jax.experimental import pallas as pl
from jax.experimental.pallas import tpu as pltpu
from jax.experimental.pallas import tpu_sc as plsc

B, C, H, W = 16, 96, 224, 224
NC, NS, L = 2, 16, 16
NW = NC * NS
CQ = C // 4
UNITS = 14
NBLK = UNITS * 4
VPR_W = W // L

_mesh = plsc.VectorSubcoreMesh(
    core_axis_name="c", subcore_axis_name="s", num_cores=NC, num_subcores=NS
)


@functools.partial(
    pl.kernel,
    out_type=jax.ShapeDtypeStruct((NW, 2 * L), jnp.float32),
    mesh=_mesh,
    compiler_params=pltpu.CompilerParams(
        use_tc_tiling_on_sc=True, needs_layout_passes=False
    ),
    scratch_types=[
        pltpu.VMEM((CQ, 8, W), jnp.float32),
        pltpu.VMEM((CQ, 8, W), jnp.float32),
        pltpu.VMEM((8, W), jnp.int32),
        pltpu.VMEM((8, W), jnp.float32),
        pltpu.VMEM((2 * L,), jnp.float32),
        pltpu.SemaphoreType.DMA,
        pltpu.SemaphoreType.DMA,
    ],
)
def _traloss_sc(yhat_hbm, label_hbm, mask_hbm, parts_hbm,
                y0_v, y1_v, lab_v, m_v, part_v, sem0, sem1):
    c = lax.axis_index("c")
    s = lax.axis_index("s")
    wid = c * NS + s
    b = lax.shift_right_logical(wid, 1)
    hb0 = UNITS * (wid & 1)
    lane = lax.iota(jnp.int32, L)
    zero = jnp.zeros((L,), jnp.float32)

    def src(g):
        h8 = (hb0 + lax.shift_right_logical(g, 2)) * 8
        c0 = CQ * (g & 3)
        return yhat_hbm.at[b, pl.ds(c0, CQ), pl.ds(h8, 8)]

    def fire(g, buf, sem):
        pltpu.async_copy(src(g), buf, sem)

    def drain(g, buf, sem):
        pltpu.make_async_copy(src(g), buf, sem).wait()

    def compute(g, buf, carry):
        an, ad = carry
        c0 = CQ * (g & 3)
        first_q = (g & 3) == 0
        for r in range(8):
            for k in range(VPR_W):
                cs = k * L
                lab = lab_v[r, pl.ds(cs, L)]
                cc = lab - c0
                inr = (cc >= 0) & (cc < CQ)
                cidx = jnp.clip(cc, 0, CQ - 1)
                hidx = jnp.full((L,), r, jnp.int32)
                widx = cs + lane
                g_val = plsc.load_gather(buf, [cidx, hidx, widx])
                m = m_v[r, pl.ds(cs, L)]
                an = an + jnp.where(inr, g_val, 0.0) * m
                ad = ad + jnp.where(first_q, m, 0.0)
        return an, ad

    def unit_prefetch_labels(g):
        h8 = (hb0 + lax.shift_right_logical(g, 2)) * 8
        pltpu.sync_copy(label_hbm.at[b, 0, pl.ds(h8, 8)], lab_v)
        pltpu.sync_copy(mask_hbm.at[b, 0, pl.ds(h8, 8)], m_v)

    fire(0, y0_v, sem0)

    def pair_body(j, carry):
        g0 = 2 * j

        @pl.when(g0 + 1 < NBLK)
        def _():
            fire(g0 + 1, y1_v, sem1)

        unit_prefetch_labels(g0)
        drain(g0, y0_v, sem0)
        carry = compute(g0, y0_v, carry)

        @pl.when(g0 + 2 < NBLK)
        def _():
            fire(g0 + 2, y0_v, sem0)

        unit_prefetch_labels(g0 + 1)
        drain(g0 + 1, y1_v, sem1)
        carry = compute(g0 + 1, y1_v, carry)
        return carry

    an, ad = lax.fori_loop(0, NBLK // 2, pair_body, (zero, zero))

    part_v[pl.ds(0, L)] = an
    part_v[pl.ds(L, L)] = ad
    pltpu.sync_copy(part_v, parts_hbm.at[wid])


def _fold_body(parts_ref, o_ref):
    x = parts_ref[...]
    rn = jnp.sum(x[:, :L], axis=1)
    rd = jnp.sum(x[:, L:], axis=1)
    row = lax.broadcasted_iota(jnp.int32, (B, NW), 1)
    samp = lax.broadcasted_iota(jnp.int32, (B, NW), 0)
    sel = jnp.where(row // 2 == samp, 1.0, 0.0)
    num = jnp.sum(sel * rn[None, :], axis=1)
    den = jnp.sum(sel * rd[None, :], axis=1)
    o_ref[0, 0] = jnp.sum(num / den) * (1.0 / B)


_fold = pl.pallas_call(
    _fold_body,
    out_shape=jax.ShapeDtypeStruct((1, 1), jnp.float32),
    out_specs=pl.BlockSpec(memory_space=pltpu.SMEM),
)


def kernel(y_hat, label, mask):
    parts = _traloss_sc(y_hat, label.astype(jnp.int32), mask)
    return _fold(parts)[0, 0]
